# baseline (speedup 1.0000x reference)
import jax
import jax.numpy as jnp
from jax import lax
from jax.experimental import pallas as pl
from jax.experimental.pallas import tpu as pltpu
from jax.experimental.pallas import tpu_sc as plsc

VOCAB = 1000000
EMB_DIM = 64
MASK_DIM = 50
B = 4096
L = 200
N = B * L

G = 128
KJ = 4
C = G * KJ
NW = 32
ROWS_PER_W = N // (G * NW)
STEPS = ROWS_PER_W // KJ
TB = B // NW
LANES = 16


def _sc_body(sent_hbm, mask_hbm, target_hbm, wtab_hbm, mtab_hbm,
             out_hbm, tout_hbm,
             idx_v, midx_v, tidx_v, wrows, mrows, trows, mtab_v, last2,
             wsem, msem, tsem):
    nc = plsc.get_sparse_core_info().num_cores
    wid = lax.axis_index("s") * nc + lax.axis_index("c")

    pltpu.sync_copy(target_hbm.at[wid], tidx_v)
    pltpu.async_copy(wtab_hbm.at[tidx_v], trows, tsem).wait()
    pltpu.sync_copy(trows, tout_hbm.at[pl.ds(wid * TB, TB)])

    pltpu.sync_copy(mtab_hbm, mtab_v)

    lane_iota = lax.iota(jnp.int32, LANES)
    col48 = jnp.full((LANES,), 48, jnp.int32)
    col49 = jnp.full((LANES,), 49, jnp.int32)
    zero = jnp.zeros((LANES,), jnp.int32)
    one = jnp.ones((LANES,), jnp.int32)

    w0 = wid * ROWS_PER_W

    def step(i):
        r0 = w0 + i * KJ
        base = r0 * G
        pltpu.sync_copy(sent_hbm.at[pl.ds(r0, KJ)], idx_v)
        pltpu.sync_copy(mask_hbm.at[pl.ds(r0, KJ)], midx_v)
        cps = []
        for j in range(KJ):
            cps.append(pltpu.async_copy(
                wtab_hbm.at[idx_v.at[j]], wrows.at[pl.ds(j * G, G)], wsem))
            cps.append(pltpu.async_copy(
                mtab_hbm.at[midx_v.at[j]], mrows.at[pl.ds(j * G, G)], msem))
        for j in range(KJ):
            for g in range(G // LANES):
                m = midx_v[j, pl.ds(g * LANES, LANES)]
                rows16 = jnp.full((LANES,), j * G + g * LANES,
                                  jnp.int32) + lane_iota
                va = plsc.load_gather(mtab_v, [m, col48])
                vb = plsc.load_gather(mtab_v, [m, col49])
                plsc.store_scatter(last2, [rows16, zero], va)
                plsc.store_scatter(last2, [rows16, one], vb)
        for cp in cps:
            cp.wait()
        pltpu.sync_copy(wrows, out_hbm.at[pl.ds(base, C), pl.ds(0, EMB_DIM)])
        pltpu.sync_copy(mrows.at[:, pl.ds(0, 48)],
                        out_hbm.at[pl.ds(base, C), pl.ds(EMB_DIM, 48)])
        pltpu.sync_copy(last2, out_hbm.at[pl.ds(base, C), pl.ds(112, 2)])

    pl.loop(0, STEPS)(step)


@jax.jit
def _sc_cat(sent2d, mask2d, target2d, word_table, mask_table_pad):
    mesh = plsc.VectorSubcoreMesh(core_axis_name="c", subcore_axis_name="s")
    f = pl.kernel(
        _sc_body,
        out_type=(
            jax.ShapeDtypeStruct((N, EMB_DIM + MASK_DIM), jnp.float32),
            jax.ShapeDtypeStruct((B, EMB_DIM), jnp.float32),
        ),
        mesh=mesh,
        compiler_params=pltpu.CompilerParams(use_tc_tiling_on_sc=False, needs_layout_passes=False),
        scratch_types=[
            pltpu.VMEM((KJ, G), jnp.int32),
            pltpu.VMEM((KJ, G), jnp.int32),
            pltpu.VMEM((TB,), jnp.int32),
            pltpu.VMEM((C, EMB_DIM), jnp.float32),
            pltpu.VMEM((C, EMB_DIM), jnp.float32),
            pltpu.VMEM((TB, EMB_DIM), jnp.float32),
            pltpu.VMEM((2, EMB_DIM), jnp.float32),
            pltpu.VMEM((C, 2), jnp.float32),
            pltpu.SemaphoreType.DMA,
            pltpu.SemaphoreType.DMA,
            pltpu.SemaphoreType.DMA,
        ],
    )
    return f(sent2d, mask2d, target2d, word_table, mask_table_pad)


def kernel(sent, mask, target, word_table, mask_table):
    sent2d = sent.reshape(N // G, G).astype(jnp.int32)
    mask2d = mask.reshape(N // G, G).astype(jnp.int32)
    target2d = target.reshape(NW, TB).astype(jnp.int32)
    mtab_pad = jnp.pad(mask_table, ((0, 0), (0, EMB_DIM - MASK_DIM)))
    out, tout = _sc_cat(sent2d, mask2d, target2d, word_table, mtab_pad)
    return out.reshape(B, L, EMB_DIM + MASK_DIM), tout


# speedup vs baseline: 9.6505x; 9.6505x over previous
import jax
import jax.numpy as jnp
from jax import lax
from jax.experimental import pallas as pl
from jax.experimental.pallas import tpu as pltpu
from jax.experimental.pallas import tpu_sc as plsc

VOCAB = 1000000
EMB_DIM = 64
MASK_DIM = 50
B = 4096
L = 200
N = B * L

G = 128
KJ = 4
C = G * KJ
NW = 32
ROWS_PER_W = N // (G * NW)
STEPS = ROWS_PER_W // KJ
TB = B // NW
LANES = 16


def _sc_body(sent_hbm, mask_hbm, target_hbm, wtab_hbm, mtab_hbm,
             out_hbm, tout_hbm,
             idx_v, midx_v, tidx_v, wrows, mrows, trows, mtab_v, mtab_sh,
             last2, wsem, msem, tsem):
    nc = plsc.get_sparse_core_info().num_cores
    sid = lax.axis_index("s")
    wid = sid * nc + lax.axis_index("c")

    pltpu.sync_copy(mtab_hbm, mtab_v)
    @pl.when(sid == 0)
    def _stage():
        pltpu.sync_copy(mtab_v, mtab_sh)
    plsc.subcore_barrier()

    pltpu.sync_copy(target_hbm.at[wid], tidx_v)
    pltpu.async_copy(wtab_hbm.at[tidx_v], trows, tsem).wait()
    pltpu.sync_copy(trows, tout_hbm.at[pl.ds(wid * TB, TB)])

    lane_iota = lax.iota(jnp.int32, LANES)
    col48 = jnp.full((LANES,), 48, jnp.int32)
    col49 = jnp.full((LANES,), 49, jnp.int32)
    zero = jnp.zeros((LANES,), jnp.int32)
    one = jnp.ones((LANES,), jnp.int32)

    w0 = wid * ROWS_PER_W

    def step(i):
        r0 = w0 + i * KJ
        base = r0 * G
        pltpu.sync_copy(sent_hbm.at[pl.ds(r0, KJ)], idx_v)
        pltpu.sync_copy(mask_hbm.at[pl.ds(r0, KJ)], midx_v)
        cps = []
        for j in range(KJ):
            cps.append(pltpu.async_copy(
                wtab_hbm.at[idx_v.at[j]], wrows.at[pl.ds(j * G, G)], wsem))
            cps.append(pltpu.async_copy(
                mtab_sh.at[midx_v.at[j]], mrows.at[pl.ds(j * G, G)], msem))
        for j in range(KJ):
            for g in range(G // LANES):
                m = midx_v[j, pl.ds(g * LANES, LANES)]
                rows16 = jnp.full((LANES,), j * G + g * LANES,
                                  jnp.int32) + lane_iota
                va = plsc.load_gather(mtab_v, [m, col48])
                vb = plsc.load_gather(mtab_v, [m, col49])
                plsc.store_scatter(last2, [rows16, zero], va)
                plsc.store_scatter(last2, [rows16, one], vb)
        for cp in cps:
            cp.wait()
        pltpu.sync_copy(wrows, out_hbm.at[pl.ds(base, C), pl.ds(0, EMB_DIM)])
        pltpu.sync_copy(mrows.at[:, pl.ds(0, 48)],
                        out_hbm.at[pl.ds(base, C), pl.ds(EMB_DIM, 48)])
        pltpu.sync_copy(last2, out_hbm.at[pl.ds(base, C), pl.ds(112, 2)])

    pl.loop(0, STEPS)(step)


@jax.jit
def _sc_cat(sent2d, mask2d, target2d, word_table, mask_table_pad):
    mesh = plsc.VectorSubcoreMesh(core_axis_name="c", subcore_axis_name="s")
    f = pl.kernel(
        _sc_body,
        out_type=(
            jax.ShapeDtypeStruct((N, EMB_DIM + MASK_DIM), jnp.float32),
            jax.ShapeDtypeStruct((B, EMB_DIM), jnp.float32),
        ),
        mesh=mesh,
        compiler_params=pltpu.CompilerParams(use_tc_tiling_on_sc=False, needs_layout_passes=False),
        scratch_types=[
            pltpu.VMEM((KJ, G), jnp.int32),
            pltpu.VMEM((KJ, G), jnp.int32),
            pltpu.VMEM((TB,), jnp.int32),
            pltpu.VMEM((C, EMB_DIM), jnp.float32),
            pltpu.VMEM((C, EMB_DIM), jnp.float32),
            pltpu.VMEM((TB, EMB_DIM), jnp.float32),
            pltpu.VMEM((2, EMB_DIM), jnp.float32),
            pltpu.VMEM_SHARED((2, EMB_DIM), jnp.float32),
            pltpu.VMEM((C, 2), jnp.float32),
            pltpu.SemaphoreType.DMA,
            pltpu.SemaphoreType.DMA,
            pltpu.SemaphoreType.DMA,
        ],
    )
    return f(sent2d, mask2d, target2d, word_table, mask_table_pad)


def kernel(sent, mask, target, word_table, mask_table):
    sent2d = sent.reshape(N // G, G).astype(jnp.int32)
    mask2d = mask.reshape(N // G, G).astype(jnp.int32)
    target2d = target.reshape(NW, TB).astype(jnp.int32)
    mtab_pad = jnp.pad(mask_table, ((0, 0), (0, EMB_DIM - MASK_DIM)))
    out, tout = _sc_cat(sent2d, mask2d, target2d, word_table, mtab_pad)
    return out.reshape(B, L, EMB_DIM + MASK_DIM), tout


# speedup vs baseline: 10.4352x; 1.0813x over previous
import jax
import jax.numpy as jnp
from jax import lax
from jax.experimental import pallas as pl
from jax.experimental.pallas import tpu as pltpu
from jax.experimental.pallas import tpu_sc as plsc

VOCAB = 1000000
EMB_DIM = 64
MASK_DIM = 50
B = 4096
L = 200
N = B * L

G = 128
KJ = 4
C = G * KJ
NW = 32
ROWS_PER_W = N // (G * NW)
STEPS = ROWS_PER_W // KJ
TB = B // NW
LANES = 16
MW = 48


def _sc_body(sent_hbm, mask_hbm, target_hbm, wtab_hbm, mtab_hbm,
             out_hbm, tout_hbm,
             idx_v, midx_v, tidx_v, wrows, mrows, trows, mtab_v, mtab_sh,
             last2, wsems, msems, tsem):
    nc = plsc.get_sparse_core_info().num_cores
    sid = lax.axis_index("s")
    wid = sid * nc + lax.axis_index("c")

    pltpu.sync_copy(mtab_hbm, mtab_v)

    @pl.when(sid == 0)
    def _stage():
        pltpu.sync_copy(mtab_v.at[:, pl.ds(0, MW)], mtab_sh)

    plsc.subcore_barrier()

    pltpu.sync_copy(target_hbm.at[wid], tidx_v)
    pltpu.async_copy(wtab_hbm.at[tidx_v], trows, tsem).wait()
    pltpu.sync_copy(trows, tout_hbm.at[pl.ds(wid * TB, TB)])

    lane_iota = lax.iota(jnp.int32, LANES)
    col48 = jnp.full((LANES,), MW, jnp.int32)
    col49 = jnp.full((LANES,), MW + 1, jnp.int32)
    zero = jnp.zeros((LANES,), jnp.int32)
    one = jnp.ones((LANES,), jnp.int32)

    w0 = wid * ROWS_PER_W

    def load_and_fire(i, p):
        r0 = w0 + i * KJ
        pltpu.sync_copy(sent_hbm.at[pl.ds(r0, KJ)], idx_v[p])
        pltpu.sync_copy(mask_hbm.at[pl.ds(r0, KJ)], midx_v[p])
        for j in range(KJ):
            pltpu.async_copy(wtab_hbm.at[idx_v[p].at[j]],
                             wrows[p].at[pl.ds(j * G, G)], wsems[p])
            pltpu.async_copy(mtab_sh.at[midx_v[p].at[j]],
                             mrows[p].at[pl.ds(j * G, G)], msems[p])

    def finish(i, p):
        for j in range(KJ):
            for g in range(G // LANES):
                m = midx_v[p][j, pl.ds(g * LANES, LANES)]
                rows16 = jnp.full((LANES,), j * G + g * LANES,
                                  jnp.int32) + lane_iota
                va = plsc.load_gather(mtab_v, [m, col48])
                vb = plsc.load_gather(mtab_v, [m, col49])
                plsc.store_scatter(last2, [rows16, zero], va)
                plsc.store_scatter(last2, [rows16, one], vb)
        for j in range(KJ):
            pltpu.make_async_copy(wtab_hbm.at[idx_v[p].at[j]],
                                  wrows[p].at[pl.ds(j * G, G)],
                                  wsems[p]).wait()
            pltpu.make_async_copy(mtab_sh.at[midx_v[p].at[j]],
                                  mrows[p].at[pl.ds(j * G, G)],
                                  msems[p]).wait()
        base = (w0 + i * KJ) * G
        pltpu.sync_copy(wrows[p],
                        out_hbm.at[pl.ds(base, C), pl.ds(0, EMB_DIM)])
        pltpu.sync_copy(mrows[p],
                        out_hbm.at[pl.ds(base, C), pl.ds(EMB_DIM, MW)])
        pltpu.sync_copy(last2, out_hbm.at[pl.ds(base, C), pl.ds(112, 2)])

    load_and_fire(0, 0)

    def pair(k):
        i0 = k * 2
        @pl.when(i0 + 1 < STEPS)
        def _a():
            load_and_fire(i0 + 1, 1)
        finish(i0, 0)
        @pl.when(i0 + 2 < STEPS)
        def _b():
            load_and_fire(i0 + 2, 0)
        @pl.when(i0 + 1 < STEPS)
        def _c():
            finish(i0 + 1, 1)

    pl.loop(0, (STEPS + 1) // 2)(pair)


@jax.jit
def _sc_cat(sent2d, mask2d, target2d, word_table, mask_table_pad):
    mesh = plsc.VectorSubcoreMesh(core_axis_name="c", subcore_axis_name="s")
    f = pl.kernel(
        _sc_body,
        out_type=(
            jax.ShapeDtypeStruct((N, EMB_DIM + MASK_DIM), jnp.float32),
            jax.ShapeDtypeStruct((B, EMB_DIM), jnp.float32),
        ),
        mesh=mesh,
        compiler_params=pltpu.CompilerParams(use_tc_tiling_on_sc=False,
                                             needs_layout_passes=False),
        scratch_types=[
            [pltpu.VMEM((KJ, G), jnp.int32)] * 2,
            [pltpu.VMEM((KJ, G), jnp.int32)] * 2,
            pltpu.VMEM((TB,), jnp.int32),
            [pltpu.VMEM((C, EMB_DIM), jnp.float32)] * 2,
            [pltpu.VMEM((C, MW), jnp.float32)] * 2,
            pltpu.VMEM((TB, EMB_DIM), jnp.float32),
            pltpu.VMEM((2, EMB_DIM), jnp.float32),
            pltpu.VMEM_SHARED((2, MW), jnp.float32),
            pltpu.VMEM((C, 2), jnp.float32),
            [pltpu.SemaphoreType.DMA] * 2,
            [pltpu.SemaphoreType.DMA] * 2,
            pltpu.SemaphoreType.DMA,
        ],
    )
    return f(sent2d, mask2d, target2d, word_table, mask_table_pad)


def kernel(sent, mask, target, word_table, mask_table):
    sent2d = sent.reshape(N // G, G).astype(jnp.int32)
    mask2d = mask.reshape(N // G, G).astype(jnp.int32)
    target2d = target.reshape(NW, TB).astype(jnp.int32)
    mtab_pad = jnp.pad(mask_table, ((0, 0), (0, EMB_DIM - MASK_DIM)))
    out, tout = _sc_cat(sent2d, mask2d, target2d, word_table, mtab_pad)
    return out.reshape(B, L, EMB_DIM + MASK_DIM), tout
